# Initial kernel scaffold; baseline (speedup 1.0000x reference)
import functools

import jax
import jax.numpy as jnp
from jax import lax
from jax.experimental import pallas as pl
from jax.experimental.pallas import tpu as pltpu
from jax.experimental.pallas import tpu_sc as plsc

N = 50000
D = 128
DOUT = 64
B = 16384
S = 16
EPS = 1e-5

NC = 2
NS = 16
NW = NC * NS
BPW = B // NW
CHUNK = 128
BPC = CHUNK // S
NCHUNK = BPW * S // CHUNK
INV_S = 1.0 / S


def _sc_body(x0_hbm, h1_hbm, nidx_hbm, nb_hbm, agg_hbm, hb_hbm,
             nidx_v, nb_v, rows_v, out_v, hrows_v, sem):
    wid = lax.axis_index("s") * NC + lax.axis_index("c")
    bbase = wid * BPW
    pltpu.sync_copy(nidx_hbm.at[pl.ds(bbase * S, BPW * S)], nidx_v)
    pltpu.sync_copy(nb_hbm.at[pl.ds(bbase, BPW)], nb_v)

    def chunk_body(c, carry):
        pltpu.async_copy(
            x0_hbm.at[nidx_v.at[pl.ds(c * CHUNK, CHUNK)]], rows_v, sem
        ).wait()

        def red_body(i, carry2):
            row0 = i * S
            for g in range(D // 16):
                col = g * 16
                acc = rows_v[row0, pl.ds(col, 16)]
                for s_ in range(1, S):
                    acc = acc + rows_v[row0 + s_, pl.ds(col, 16)]
            out_v[i, pl.ds(col, 16)] = acc * jnp.float32(INV_S)
            return carry2

        lax.fori_loop(0, BPC, red_body, 0, unroll=True)
        pltpu.sync_copy(out_v, agg_hbm.at[pl.ds(bbase + c * BPC, BPC)])
        return carry

    lax.fori_loop(0, NCHUNK, chunk_body, 0)

    def hb_body(c, carry):
        pltpu.async_copy(
            h1_hbm.at[nb_v.at[pl.ds(c * CHUNK, CHUNK)]], hrows_v, sem
        ).wait()
        pltpu.sync_copy(hrows_v, hb_hbm.at[pl.ds(bbase + c * CHUNK, CHUNK)])
        return carry

    lax.fori_loop(0, BPW // CHUNK, hb_body, 0)


_sc_gather = functools.partial(
    pl.kernel,
    out_type=[
        jax.ShapeDtypeStruct((B, D), jnp.float32),
        jax.ShapeDtypeStruct((B, D), jnp.float32),
    ],
    mesh=plsc.VectorSubcoreMesh(core_axis_name="c", subcore_axis_name="s"),
    scratch_types=[
        pltpu.VMEM((BPW * S,), jnp.int32),
        pltpu.VMEM((BPW,), jnp.int32),
        pltpu.VMEM((CHUNK, D), jnp.float32),
        pltpu.VMEM((BPC, D), jnp.float32),
        pltpu.VMEM((CHUNK, D), jnp.float32),
        pltpu.SemaphoreType.DMA,
    ],
)(_sc_body)


def _tc_body(agg_ref, hb_ref, w2a_ref, w2b_ref, b2_ref, g2_ref, be2_ref,
             wout_ref, bout_ref, out_ref):
    h = jnp.dot(agg_ref[...], w2a_ref[...], preferred_element_type=jnp.float32)
    h = h + jnp.dot(hb_ref[...], w2b_ref[...], preferred_element_type=jnp.float32)
    h = h + b2_ref[...]
    h = jnp.maximum(h, 0.0)
    mu = jnp.mean(h, axis=1, keepdims=True)
    d = h - mu
    var = jnp.mean(d * d, axis=1, keepdims=True)
    h = d * lax.rsqrt(var + EPS) * g2_ref[...] + be2_ref[...]
    logits = jnp.dot(h, wout_ref[...], preferred_element_type=jnp.float32)
    logits = logits + bout_ref[...]
    m = jnp.max(logits, axis=1, keepdims=True)
    e = jnp.exp(logits - m)
    out_ref[...] = e / jnp.sum(e, axis=1, keepdims=True)


TC_BLK = 2048


def _tc_dense(agg, hb, w2a, w2b, b2, g2, be2, wout, bout):
    grid = (B // TC_BLK,)
    row_blk = pl.BlockSpec((TC_BLK, D), lambda i: (i, 0))

    def rep(shape):
        return pl.BlockSpec(shape, lambda i: (0, 0))

    return pl.pallas_call(
        _tc_body,
        grid=grid,
        in_specs=[
            row_blk,
            row_blk,
            rep((D, D)),
            rep((D, D)),
            rep((1, D)),
            rep((1, D)),
            rep((1, D)),
            rep((D, DOUT)),
            rep((1, DOUT)),
        ],
        out_specs=pl.BlockSpec((TC_BLK, DOUT), lambda i: (i, 0)),
        out_shape=jax.ShapeDtypeStruct((B, DOUT), jnp.float32),
    )(agg, hb, w2a, w2b, b2, g2, be2, wout, bout)


def kernel(x0, h1, node_batch, neigh_idx_1, neigh_idx_2,
           W1, b1, g1, be1, W2, b2, g2, be2, Wout, bout):
    del neigh_idx_1, W1, b1, g1, be1
    nidx = neigh_idx_2.reshape(-1).astype(jnp.int32)
    nb = node_batch.astype(jnp.int32)
    agg, hb = _sc_gather(x0, h1, nidx, nb)
    return _tc_dense(
        agg, hb, W2[:D], W2[D:], b2.reshape(1, D), g2.reshape(1, D),
        be2.reshape(1, D), Wout, bout.reshape(1, DOUT))


# speedup vs baseline: 6.0592x; 6.0592x over previous
import functools

import jax
import jax.numpy as jnp
from jax import lax
from jax.experimental import pallas as pl
from jax.experimental.pallas import tpu as pltpu
from jax.experimental.pallas import tpu_sc as plsc

N = 50000
D = 128
DOUT = 64
B = 16384
S = 16
EPS = 1e-5

NC = 2
NS = 16
NW = NC * NS
BPW = B // NW
CHUNK = 128
BPC = CHUNK // S
NCHUNK = BPW * S // CHUNK
INV_S = 1.0 / S


def _sc_body(x0_hbm, h1_hbm, nidx_hbm, nb_hbm, agg_hbm, hb_hbm,
             nidx_v, nb_v, rows_v, out_v, hrows_v, sem):
    wid = lax.axis_index("s") * NC + lax.axis_index("c")
    bbase = wid * BPW
    pltpu.sync_copy(nidx_hbm.at[pl.ds(bbase * S, BPW * S)], nidx_v)
    pltpu.sync_copy(nb_hbm.at[pl.ds(bbase, BPW)], nb_v)

    def chunk_body(c, carry):
        pltpu.async_copy(
            x0_hbm.at[nidx_v.at[pl.ds(c * CHUNK, CHUNK)]], rows_v, sem
        ).wait()

        def red_body(i, carry2):
            row0 = i * S
            for g in range(D // 16):
                col = g * 16
                acc = rows_v[row0, pl.ds(col, 16)]
                for s_ in range(1, S):
                    acc = acc + rows_v[row0 + s_, pl.ds(col, 16)]
                out_v[i, pl.ds(col, 16)] = acc * jnp.float32(INV_S)
            return carry2

        lax.fori_loop(0, BPC, red_body, 0, unroll=True)
        pltpu.sync_copy(out_v, agg_hbm.at[pl.ds(bbase + c * BPC, BPC)])
        return carry

    lax.fori_loop(0, NCHUNK, chunk_body, 0)

    def hb_body(c, carry):
        pltpu.async_copy(
            h1_hbm.at[nb_v.at[pl.ds(c * CHUNK, CHUNK)]], hrows_v, sem
        ).wait()
        pltpu.sync_copy(hrows_v, hb_hbm.at[pl.ds(bbase + c * CHUNK, CHUNK)])
        return carry

    lax.fori_loop(0, BPW // CHUNK, hb_body, 0)


_sc_gather = functools.partial(
    pl.kernel,
    out_type=[
        jax.ShapeDtypeStruct((B, D), jnp.float32),
        jax.ShapeDtypeStruct((B, D), jnp.float32),
    ],
    mesh=plsc.VectorSubcoreMesh(core_axis_name="c", subcore_axis_name="s"),
    scratch_types=[
        pltpu.VMEM((BPW * S,), jnp.int32),
        pltpu.VMEM((BPW,), jnp.int32),
        pltpu.VMEM((CHUNK, D), jnp.float32),
        pltpu.VMEM((BPC, D), jnp.float32),
        pltpu.VMEM((CHUNK, D), jnp.float32),
        pltpu.SemaphoreType.DMA,
    ],
)(_sc_body)


def _tc_body(agg_ref, hb_ref, w2a_ref, w2b_ref, b2_ref, g2_ref, be2_ref,
             wout_ref, bout_ref, out_ref):
    h = jnp.dot(agg_ref[...], w2a_ref[...], preferred_element_type=jnp.float32)
    h = h + jnp.dot(hb_ref[...], w2b_ref[...], preferred_element_type=jnp.float32)
    h = h + b2_ref[...]
    h = jnp.maximum(h, 0.0)
    mu = jnp.mean(h, axis=1, keepdims=True)
    d = h - mu
    var = jnp.mean(d * d, axis=1, keepdims=True)
    h = d * lax.rsqrt(var + EPS) * g2_ref[...] + be2_ref[...]
    logits = jnp.dot(h, wout_ref[...], preferred_element_type=jnp.float32)
    logits = logits + bout_ref[...]
    m = jnp.max(logits, axis=1, keepdims=True)
    e = jnp.exp(logits - m)
    out_ref[...] = e / jnp.sum(e, axis=1, keepdims=True)


TC_BLK = 2048


def _tc_dense(agg, hb, w2a, w2b, b2, g2, be2, wout, bout):
    grid = (B // TC_BLK,)
    row_blk = pl.BlockSpec((TC_BLK, D), lambda i: (i, 0))

    def rep(shape):
        return pl.BlockSpec(shape, lambda i: (0, 0))

    return pl.pallas_call(
        _tc_body,
        grid=grid,
        in_specs=[
            row_blk,
            row_blk,
            rep((D, D)),
            rep((D, D)),
            rep((1, D)),
            rep((1, D)),
            rep((1, D)),
            rep((D, DOUT)),
            rep((1, DOUT)),
        ],
        out_specs=pl.BlockSpec((TC_BLK, DOUT), lambda i: (i, 0)),
        out_shape=jax.ShapeDtypeStruct((B, DOUT), jnp.float32),
    )(agg, hb, w2a, w2b, b2, g2, be2, wout, bout)


def kernel(x0, h1, node_batch, neigh_idx_1, neigh_idx_2,
           W1, b1, g1, be1, W2, b2, g2, be2, Wout, bout):
    del neigh_idx_1, W1, b1, g1, be1
    nidx = neigh_idx_2.reshape(-1).astype(jnp.int32)
    nb = node_batch.astype(jnp.int32)
    agg, hb = _sc_gather(x0, h1, nidx, nb)
    return _tc_dense(
        agg, hb, W2[:D], W2[D:], b2.reshape(1, D), g2.reshape(1, D),
        be2.reshape(1, D), Wout, bout.reshape(1, DOUT))


# speedup vs baseline: 6.1708x; 1.0184x over previous
import functools

import jax
import jax.numpy as jnp
from jax import lax
from jax.experimental import pallas as pl
from jax.experimental.pallas import tpu as pltpu
from jax.experimental.pallas import tpu_sc as plsc

N = 50000
D = 128
DOUT = 64
B = 16384
S = 16
EPS = 1e-5

NC = 2
NS = 16
NW = NC * NS
BPW = B // NW
CHUNK = 128
BPC = CHUNK // S
NCHUNK = BPW * S // CHUNK
INV_S = 1.0 / S


def _sc_body(x0_hbm, h1_hbm, nidx_hbm, nb_hbm, agg_hbm, hb_hbm,
             nidx_v, nb_v, rows_a, rows_b, agg_v, sem_a, sem_b):
    wid = lax.axis_index("s") * NC + lax.axis_index("c")
    bbase = wid * BPW
    pltpu.sync_copy(nidx_hbm.at[pl.ds(bbase * S, BPW * S)], nidx_v)
    pltpu.sync_copy(nb_hbm.at[pl.ds(bbase, BPW)], nb_v)

    def start(c, buf, sem):
        return pltpu.async_copy(
            x0_hbm.at[nidx_v.at[pl.ds(c * CHUNK, CHUNK)]], buf, sem)

    def reduce_chunk(c, buf):
        def red_body(i, carry):
            row0 = i * S
            for g in range(D // 16):
                col = g * 16
                acc = buf[row0, pl.ds(col, 16)]
                for s_ in range(1, S):
                    acc = acc + buf[row0 + s_, pl.ds(col, 16)]
                agg_v[c * BPC + i, pl.ds(col, 16)] = acc * jnp.float32(INV_S)
            return carry

        lax.fori_loop(0, BPC, red_body, 0, unroll=True)

    def wait(buf, sem):
        pltpu.make_async_copy(x0_hbm.at[nidx_v.at[pl.ds(0, CHUNK)]],
                              buf, sem).wait()

    start(0, rows_a, sem_a)

    def pipe_body(k, carry):
        c = 2 * k
        start(c + 1, rows_b, sem_b)
        wait(rows_a, sem_a)
        reduce_chunk(c, rows_a)
        start(c + 2, rows_a, sem_a)
        wait(rows_b, sem_b)
        reduce_chunk(c + 1, rows_b)
        return carry

    lax.fori_loop(0, NCHUNK // 2 - 1, pipe_body, 0)
    start(NCHUNK - 1, rows_b, sem_b)
    wait(rows_a, sem_a)
    reduce_chunk(NCHUNK - 2, rows_a)
    wait(rows_b, sem_b)
    reduce_chunk(NCHUNK - 1, rows_b)
    pltpu.sync_copy(agg_v, agg_hbm.at[pl.ds(bbase, BPW)])

    HCH = BPW // CHUNK
    def hstart(c, buf, sem):
        return pltpu.async_copy(
            h1_hbm.at[nb_v.at[pl.ds(c * CHUNK, CHUNK)]], buf, sem)

    hstart(0, rows_a, sem_a)
    for c in range(HCH):
        buf, sem = (rows_a, sem_a) if c % 2 == 0 else (rows_b, sem_b)
        if c + 1 < HCH:
            nbuf, nsem = (rows_a, sem_a) if (c + 1) % 2 == 0 else (rows_b, sem_b)
            hstart(c + 1, nbuf, nsem)
        pltpu.make_async_copy(h1_hbm.at[nb_v.at[pl.ds(0, CHUNK)]],
                              buf, sem).wait()
        pltpu.sync_copy(buf, hb_hbm.at[pl.ds(bbase + c * CHUNK, CHUNK)])


_sc_gather = functools.partial(
    pl.kernel,
    out_type=[
        jax.ShapeDtypeStruct((B, D), jnp.float32),
        jax.ShapeDtypeStruct((B, D), jnp.float32),
    ],
    mesh=plsc.VectorSubcoreMesh(core_axis_name="c", subcore_axis_name="s"),
    scratch_types=[
        pltpu.VMEM((BPW * S,), jnp.int32),
        pltpu.VMEM((BPW,), jnp.int32),
        pltpu.VMEM((CHUNK, D), jnp.float32),
        pltpu.VMEM((CHUNK, D), jnp.float32),
        pltpu.VMEM((BPW, D), jnp.float32),
        pltpu.SemaphoreType.DMA,
        pltpu.SemaphoreType.DMA,
    ],
)(_sc_body)


def _tc_body(agg_ref, hb_ref, w2a_ref, w2b_ref, b2_ref, g2_ref, be2_ref,
             wout_ref, bout_ref, out_ref):
    h = jnp.dot(agg_ref[...], w2a_ref[...], preferred_element_type=jnp.float32)
    h = h + jnp.dot(hb_ref[...], w2b_ref[...], preferred_element_type=jnp.float32)
    h = h + b2_ref[...]
    h = jnp.maximum(h, 0.0)
    mu = jnp.mean(h, axis=1, keepdims=True)
    d = h - mu
    var = jnp.mean(d * d, axis=1, keepdims=True)
    h = d * lax.rsqrt(var + EPS) * g2_ref[...] + be2_ref[...]
    logits = jnp.dot(h, wout_ref[...], preferred_element_type=jnp.float32)
    logits = logits + bout_ref[...]
    m = jnp.max(logits, axis=1, keepdims=True)
    e = jnp.exp(logits - m)
    out_ref[...] = e / jnp.sum(e, axis=1, keepdims=True)


TC_BLK = 2048


def _tc_dense(agg, hb, w2a, w2b, b2, g2, be2, wout, bout):
    grid = (B // TC_BLK,)
    row_blk = pl.BlockSpec((TC_BLK, D), lambda i: (i, 0))

    def rep(shape):
        return pl.BlockSpec(shape, lambda i: (0, 0))

    return pl.pallas_call(
        _tc_body,
        grid=grid,
        in_specs=[
            row_blk,
            row_blk,
            rep((D, D)),
            rep((D, D)),
            rep((1, D)),
            rep((1, D)),
            rep((1, D)),
            rep((D, DOUT)),
            rep((1, DOUT)),
        ],
        out_specs=pl.BlockSpec((TC_BLK, DOUT), lambda i: (i, 0)),
        out_shape=jax.ShapeDtypeStruct((B, DOUT), jnp.float32),
    )(agg, hb, w2a, w2b, b2, g2, be2, wout, bout)


def kernel(x0, h1, node_batch, neigh_idx_1, neigh_idx_2,
           W1, b1, g1, be1, W2, b2, g2, be2, Wout, bout):
    del neigh_idx_1, W1, b1, g1, be1
    nidx = neigh_idx_2.reshape(-1).astype(jnp.int32)
    nb = node_batch.astype(jnp.int32)
    agg, hb = _sc_gather(x0, h1, nidx, nb)
    return _tc_dense(
        agg, hb, W2[:D], W2[D:], b2.reshape(1, D), g2.reshape(1, D),
        be2.reshape(1, D), Wout, bout.reshape(1, DOUT))


# speedup vs baseline: 16.5180x; 2.6768x over previous
import functools

import jax
import jax.numpy as jnp
from jax import lax
from jax.experimental import pallas as pl
from jax.experimental.pallas import tpu as pltpu
from jax.experimental.pallas import tpu_sc as plsc

N = 50000
D = 128
DOUT = 64
B = 16384
S = 16
EPS = 1e-5

NC = 2
NS = 16
NW = NC * NS
BPW = B // NW
CHUNK = 128
BPC = CHUNK // S
NCHUNK = BPW * S // CHUNK
INV_S = 1.0 / S


NACC = BPW // CHUNK


def _sc_body(x0_hbm, h1_hbm, nidxt_hbm, nb_hbm, agg_hbm, hb_hbm,
             nidxt_v, nb_v, acc_vs, sems, hsem):
    wid = lax.axis_index("s") * NC + lax.axis_index("c")
    bbase = wid * BPW
    for s_ in range(S):
        pltpu.sync_copy(nidxt_hbm.at[pl.ds(s_ * B + bbase, BPW)],
                        nidxt_v.at[pl.ds(s_ * BPW, BPW)])
    pltpu.sync_copy(nb_hbm.at[pl.ds(bbase, BPW)], nb_v)

    for s_ in range(S):
        for c in range(NACC):
            idx = nidxt_v.at[pl.ds(s_ * BPW + c * CHUNK, CHUNK)]
            if s_ > 0:
                pltpu.make_async_copy(
                    x0_hbm.at[idx], acc_vs[c], sems[c]).wait()
                pltpu.async_copy(x0_hbm.at[idx], acc_vs[c], sems[c],
                                 add=True)
            else:
                pltpu.async_copy(x0_hbm.at[idx], acc_vs[c], sems[c])

    for c in range(NACC):
        pltpu.make_async_copy(
            x0_hbm.at[nidxt_v.at[pl.ds(0, CHUNK)]], acc_vs[c], sems[c]
        ).wait()
        pltpu.sync_copy(acc_vs[c], agg_hbm.at[pl.ds(bbase + c * CHUNK, CHUNK)])
        pltpu.async_copy(h1_hbm.at[nb_v.at[pl.ds(c * CHUNK, CHUNK)]],
                         acc_vs[c], hsem)

    for c in range(NACC):
        pltpu.make_async_copy(
            h1_hbm.at[nb_v.at[pl.ds(0, CHUNK)]], acc_vs[c], hsem).wait()
    for c in range(NACC):
        pltpu.sync_copy(acc_vs[c], hb_hbm.at[pl.ds(bbase + c * CHUNK, CHUNK)])


def _sc_body_flat(x0_hbm, h1_hbm, nidxt_hbm, nb_hbm, agg_hbm, hb_hbm,
                  nidxt_v, nb_v, acc0, acc1, acc2, acc3,
                  sem0, sem1, sem2, sem3, hsem):
    _sc_body(x0_hbm, h1_hbm, nidxt_hbm, nb_hbm, agg_hbm, hb_hbm,
             nidxt_v, nb_v, (acc0, acc1, acc2, acc3),
             (sem0, sem1, sem2, sem3), hsem)


_sc_gather = functools.partial(
    pl.kernel,
    out_type=[
        jax.ShapeDtypeStruct((B, D), jnp.float32),
        jax.ShapeDtypeStruct((B, D), jnp.float32),
    ],
    mesh=plsc.VectorSubcoreMesh(core_axis_name="c", subcore_axis_name="s"),
    scratch_types=[
        pltpu.VMEM((BPW * S,), jnp.int32),
        pltpu.VMEM((BPW,), jnp.int32),
        pltpu.VMEM((CHUNK, D), jnp.float32),
        pltpu.VMEM((CHUNK, D), jnp.float32),
        pltpu.VMEM((CHUNK, D), jnp.float32),
        pltpu.VMEM((CHUNK, D), jnp.float32),
        pltpu.SemaphoreType.DMA,
        pltpu.SemaphoreType.DMA,
        pltpu.SemaphoreType.DMA,
        pltpu.SemaphoreType.DMA,
        pltpu.SemaphoreType.DMA,
    ],
)(_sc_body_flat)


def _tc_body(agg_ref, hb_ref, w2a_ref, w2b_ref, b2_ref, g2_ref, be2_ref,
             wout_ref, bout_ref, out_ref):
    h = jnp.dot(agg_ref[...], w2a_ref[...],
                preferred_element_type=jnp.float32) * jnp.float32(INV_S)
    h = h + jnp.dot(hb_ref[...], w2b_ref[...], preferred_element_type=jnp.float32)
    h = h + b2_ref[...]
    h = jnp.maximum(h, 0.0)
    mu = jnp.mean(h, axis=1, keepdims=True)
    d = h - mu
    var = jnp.mean(d * d, axis=1, keepdims=True)
    h = d * lax.rsqrt(var + EPS) * g2_ref[...] + be2_ref[...]
    logits = jnp.dot(h, wout_ref[...], preferred_element_type=jnp.float32)
    logits = logits + bout_ref[...]
    m = jnp.max(logits, axis=1, keepdims=True)
    e = jnp.exp(logits - m)
    out_ref[...] = e / jnp.sum(e, axis=1, keepdims=True)


TC_BLK = 2048


def _tc_dense(agg, hb, w2a, w2b, b2, g2, be2, wout, bout):
    grid = (B // TC_BLK,)
    row_blk = pl.BlockSpec((TC_BLK, D), lambda i: (i, 0))

    def rep(shape):
        return pl.BlockSpec(shape, lambda i: (0, 0))

    return pl.pallas_call(
        _tc_body,
        grid=grid,
        in_specs=[
            row_blk,
            row_blk,
            rep((D, D)),
            rep((D, D)),
            rep((1, D)),
            rep((1, D)),
            rep((1, D)),
            rep((D, DOUT)),
            rep((1, DOUT)),
        ],
        out_specs=pl.BlockSpec((TC_BLK, DOUT), lambda i: (i, 0)),
        out_shape=jax.ShapeDtypeStruct((B, DOUT), jnp.float32),
    )(agg, hb, w2a, w2b, b2, g2, be2, wout, bout)


def kernel(x0, h1, node_batch, neigh_idx_1, neigh_idx_2,
           W1, b1, g1, be1, W2, b2, g2, be2, Wout, bout):
    del neigh_idx_1, W1, b1, g1, be1
    nidxt = neigh_idx_2.astype(jnp.int32).T.reshape(-1)
    nb = node_batch.astype(jnp.int32)
    agg, hb = _sc_gather(x0, h1, nidxt, nb)
    return _tc_dense(
        agg, hb, W2[:D], W2[D:], b2.reshape(1, D), g2.reshape(1, D),
        be2.reshape(1, D), Wout, bout.reshape(1, DOUT))


# speedup vs baseline: 17.8456x; 1.0804x over previous
import functools

import jax
import jax.numpy as jnp
from jax import lax
from jax.experimental import pallas as pl
from jax.experimental.pallas import tpu as pltpu
from jax.experimental.pallas import tpu_sc as plsc

N = 50000
D = 128
DOUT = 64
B = 16384
S = 16
EPS = 1e-5

NC = 2
NS = 16
NW = NC * NS
BPW = B // NW
INV_S = 1.0 / S

CHUNK = 64
NACC = BPW // CHUNK


def _sc_body(x0_hbm, h1_hbm, nidxt_hbm, nb_hbm, agg_hbm, hb_hbm,
             nidxt_v, nb_v, acc_v, sems, hsem):
    wid = lax.axis_index("s") * NC + lax.axis_index("c")
    bbase = wid * BPW
    pltpu.sync_copy(nidxt_hbm.at[pl.ds(wid * BPW * S, BPW * S)], nidxt_v)
    pltpu.sync_copy(nb_hbm.at[pl.ds(bbase, BPW)], nb_v)

    def acc_dst(c):
        return acc_v.at[pl.ds(c * CHUNK, CHUNK)]

    for s_ in range(S):
        for c in range(NACC):
            idx = nidxt_v.at[pl.ds(s_ * BPW + c * CHUNK, CHUNK)]
            if s_ > 0:
                pltpu.make_async_copy(
                    x0_hbm.at[idx], acc_dst(c), sems.at[c]).wait()
                pltpu.async_copy(x0_hbm.at[idx], acc_dst(c), sems.at[c],
                                 add=True)
            else:
                pltpu.async_copy(x0_hbm.at[idx], acc_dst(c), sems.at[c])

    for c in range(NACC):
        pltpu.make_async_copy(
            x0_hbm.at[nidxt_v.at[pl.ds(0, CHUNK)]], acc_dst(c), sems.at[c]
        ).wait()
        pltpu.sync_copy(acc_dst(c), agg_hbm.at[pl.ds(bbase + c * CHUNK, CHUNK)])
        pltpu.async_copy(h1_hbm.at[nb_v.at[pl.ds(c * CHUNK, CHUNK)]],
                         acc_dst(c), hsem)

    for c in range(NACC):
        pltpu.make_async_copy(
            h1_hbm.at[nb_v.at[pl.ds(0, CHUNK)]], acc_dst(c), hsem).wait()
    pltpu.sync_copy(acc_v, hb_hbm.at[pl.ds(bbase, BPW)])


_sc_gather = functools.partial(
    pl.kernel,
    out_type=[
        jax.ShapeDtypeStruct((B, D), jnp.float32),
        jax.ShapeDtypeStruct((B, D), jnp.float32),
    ],
    mesh=plsc.VectorSubcoreMesh(core_axis_name="c", subcore_axis_name="s"),
    scratch_types=[
        pltpu.VMEM((BPW * S,), jnp.int32),
        pltpu.VMEM((BPW,), jnp.int32),
        pltpu.VMEM((BPW, D), jnp.float32),
        pltpu.SemaphoreType.DMA((NACC,)),
        pltpu.SemaphoreType.DMA,
    ],
)(_sc_body)


def _tc_body(agg_ref, hb_ref, w2a_ref, w2b_ref, b2_ref, g2_ref, be2_ref,
             wout_ref, bout_ref, out_ref):
    h = jnp.dot(agg_ref[...], w2a_ref[...],
                preferred_element_type=jnp.float32) * jnp.float32(INV_S)
    h = h + jnp.dot(hb_ref[...], w2b_ref[...], preferred_element_type=jnp.float32)
    h = h + b2_ref[...]
    h = jnp.maximum(h, 0.0)
    mu = jnp.mean(h, axis=1, keepdims=True)
    d = h - mu
    var = jnp.mean(d * d, axis=1, keepdims=True)
    h = d * lax.rsqrt(var + EPS) * g2_ref[...] + be2_ref[...]
    logits = jnp.dot(h, wout_ref[...], preferred_element_type=jnp.float32)
    logits = logits + bout_ref[...]
    m = jnp.max(logits, axis=1, keepdims=True)
    e = jnp.exp(logits - m)
    out_ref[...] = e / jnp.sum(e, axis=1, keepdims=True)


TC_BLK = 2048


def _tc_dense(agg, hb, w2a, w2b, b2, g2, be2, wout, bout):
    grid = (B // TC_BLK,)
    row_blk = pl.BlockSpec((TC_BLK, D), lambda i: (i, 0))

    def rep(shape):
        return pl.BlockSpec(shape, lambda i: (0, 0))

    return pl.pallas_call(
        _tc_body,
        grid=grid,
        in_specs=[
            row_blk,
            row_blk,
            rep((D, D)),
            rep((D, D)),
            rep((1, D)),
            rep((1, D)),
            rep((1, D)),
            rep((D, DOUT)),
            rep((1, DOUT)),
        ],
        out_specs=pl.BlockSpec((TC_BLK, DOUT), lambda i: (i, 0)),
        out_shape=jax.ShapeDtypeStruct((B, DOUT), jnp.float32),
    )(agg, hb, w2a, w2b, b2, g2, be2, wout, bout)


def kernel(x0, h1, node_batch, neigh_idx_1, neigh_idx_2,
           W1, b1, g1, be1, W2, b2, g2, be2, Wout, bout):
    del neigh_idx_1, W1, b1, g1, be1
    nidxt = (neigh_idx_2.astype(jnp.int32)
             .reshape(NW, BPW, S).transpose(0, 2, 1).reshape(-1))
    nb = node_batch.astype(jnp.int32)
    agg, hb = _sc_gather(x0, h1, nidxt, nb)
    return _tc_dense(
        agg, hb, W2[:D], W2[D:], b2.reshape(1, D), g2.reshape(1, D),
        be2.reshape(1, D), Wout, bout.reshape(1, DOUT))


# speedup vs baseline: 18.1006x; 1.0143x over previous
import functools

import jax
import jax.numpy as jnp
from jax import lax
from jax.experimental import pallas as pl
from jax.experimental.pallas import tpu as pltpu
from jax.experimental.pallas import tpu_sc as plsc

N = 50000
D = 128
DOUT = 64
B = 16384
S = 16
EPS = 1e-5

NC = 2
NS = 16
NW = NC * NS
BPW = B // NW
INV_S = 1.0 / S

CHUNK = 64
NACC = BPW // CHUNK


def _sc_body(x0_hbm, h1_hbm, nidxt_hbm, nb_hbm, agg_hbm, hb_hbm,
             nidxt_v, nb_v, acc_v, sems, hsem):
    wid = lax.axis_index("s") * NC + lax.axis_index("c")
    bbase = wid * BPW
    pltpu.sync_copy(nidxt_hbm.at[pl.ds(wid * BPW * S, BPW * S)], nidxt_v)
    pltpu.sync_copy(nb_hbm.at[pl.ds(bbase, BPW)], nb_v)

    def acc_dst(c):
        return acc_v.at[pl.ds(c * CHUNK, CHUNK)]

    for s_ in range(S):
        for c in range(NACC):
            idx = nidxt_v.at[pl.ds(s_ * BPW + c * CHUNK, CHUNK)]
            if s_ > 0:
                pltpu.make_async_copy(
                    x0_hbm.at[idx], acc_dst(c), sems.at[c]).wait()
                pltpu.async_copy(x0_hbm.at[idx], acc_dst(c), sems.at[c],
                                 add=True)
            else:
                pltpu.async_copy(x0_hbm.at[idx], acc_dst(c), sems.at[c])

    for c in range(NACC):
        pltpu.make_async_copy(
            x0_hbm.at[nidxt_v.at[pl.ds(0, CHUNK)]], acc_dst(c), sems.at[c]
        ).wait()
        pltpu.sync_copy(acc_dst(c), agg_hbm.at[pl.ds(bbase + c * CHUNK, CHUNK)])
        pltpu.async_copy(h1_hbm.at[nb_v.at[pl.ds(c * CHUNK, CHUNK)]],
                         acc_dst(c), hsem)

    for c in range(NACC):
        pltpu.make_async_copy(
            h1_hbm.at[nb_v.at[pl.ds(0, CHUNK)]], acc_dst(c), hsem).wait()
    pltpu.sync_copy(acc_v, hb_hbm.at[pl.ds(bbase, BPW)])


_sc_gather = functools.partial(
    pl.kernel,
    out_type=[
        jax.ShapeDtypeStruct((B, D), jnp.float32),
        jax.ShapeDtypeStruct((B, D), jnp.float32),
    ],
    mesh=plsc.VectorSubcoreMesh(core_axis_name="c", subcore_axis_name="s"),
    scratch_types=[
        pltpu.VMEM((BPW * S,), jnp.int32),
        pltpu.VMEM((BPW,), jnp.int32),
        pltpu.VMEM((BPW, D), jnp.float32),
        pltpu.SemaphoreType.DMA((NACC,)),
        pltpu.SemaphoreType.DMA,
    ],
)(_sc_body)


def _tc_body(agg_ref, hb_ref, w2a_ref, w2b_ref, b2_ref, g2_ref, be2_ref,
             wout_ref, bout_ref, out_ref):
    h = jnp.dot(agg_ref[...], w2a_ref[...],
                preferred_element_type=jnp.float32) * jnp.float32(INV_S)
    h = h + jnp.dot(hb_ref[...], w2b_ref[...], preferred_element_type=jnp.float32)
    h = h + b2_ref[...]
    h = jnp.maximum(h, 0.0)
    mu = jnp.mean(h, axis=1, keepdims=True)
    d = h - mu
    var = jnp.mean(d * d, axis=1, keepdims=True)
    h = d * lax.rsqrt(var + EPS) * g2_ref[...] + be2_ref[...]
    logits = jnp.dot(h, wout_ref[...], preferred_element_type=jnp.float32)
    logits = logits + bout_ref[...]
    m = jnp.max(logits, axis=1, keepdims=True)
    e = jnp.exp(logits - m)
    out_ref[...] = e / jnp.sum(e, axis=1, keepdims=True)


TC_BLK = 4096


def _tc_dense(agg, hb, w2a, w2b, b2, g2, be2, wout, bout):
    grid = (B // TC_BLK,)
    row_blk = pl.BlockSpec((TC_BLK, D), lambda i: (i, 0))

    def rep(shape):
        return pl.BlockSpec(shape, lambda i: (0, 0))

    return pl.pallas_call(
        _tc_body,
        grid=grid,
        in_specs=[
            row_blk,
            row_blk,
            rep((D, D)),
            rep((D, D)),
            rep((1, D)),
            rep((1, D)),
            rep((1, D)),
            rep((D, DOUT)),
            rep((1, DOUT)),
        ],
        out_specs=pl.BlockSpec((TC_BLK, DOUT), lambda i: (i, 0)),
        out_shape=jax.ShapeDtypeStruct((B, DOUT), jnp.float32),
    )(agg, hb, w2a, w2b, b2, g2, be2, wout, bout)


def kernel(x0, h1, node_batch, neigh_idx_1, neigh_idx_2,
           W1, b1, g1, be1, W2, b2, g2, be2, Wout, bout):
    del neigh_idx_1, W1, b1, g1, be1
    nidxt = (neigh_idx_2.astype(jnp.int32)
             .reshape(NW, BPW, S).transpose(0, 2, 1).reshape(-1))
    nb = node_batch.astype(jnp.int32)
    agg, hb = _sc_gather(x0, h1, nidxt, nb)
    return _tc_dense(
        agg, hb, W2[:D], W2[D:], b2.reshape(1, D), g2.reshape(1, D),
        be2.reshape(1, D), Wout, bout.reshape(1, DOUT))
